# baseline (speedup 1.0000x reference)
import functools

import jax
import jax.numpy as jnp
from jax import lax
from jax.experimental import pallas as pl
from jax.experimental.pallas import tpu as pltpu
from jax.experimental.pallas import tpu_sc as plsc

BATCH = 2
NPTS = 2048
KNN = 16
DIM = 256
HDIM = DIM // 2
PDIM = 64
XPAD = 16
XWIDE = 128

ROWS_A = 512
ROWS_C = 256

SC_CORES = 2
SC_SUBCORES = 16
SC_WORKERS = SC_CORES * SC_SUBCORES
HALF = NPTS // 2
PAIRS = HALF * KNN
PAIRS_PER_WORKER = PAIRS // SC_WORKERS
GATHER_CHUNK = 64
N_CHUNKS = PAIRS_PER_WORKER // GATHER_CHUNK

_NT = (((1,), (1,)), ((), ()))


def _mmT(a, b_ref):
    return lax.dot_general(a, b_ref[...], dimension_numbers=_NT,
                           preferred_element_type=jnp.float32)


def _mmTb(a, b_ref):
    return lax.dot_general(a.astype(jnp.bfloat16), b_ref[...],
                           dimension_numbers=_NT,
                           preferred_element_type=jnp.float32)


def _pack_bf16(v):
    lo = lax.bitcast_convert_type(v[:, :HDIM].astype(jnp.bfloat16), jnp.uint16)
    hi = lax.bitcast_convert_type(v[:, HDIM:].astype(jnp.bfloat16), jnp.uint16)
    word = lo.astype(jnp.uint32) | (hi.astype(jnp.uint32) << 16)
    return lax.bitcast_convert_type(word, jnp.int32)


def _unpack_bf16(w):
    lo = lax.bitcast_convert_type(w << 16, jnp.float32)
    hi = lax.bitcast_convert_type(w & jnp.int32(-65536), jnp.float32)
    return lo, hi


def _stage1_body(x_ref, xall_ref, inf_ref,
                 fc1w, fc1b, phiw, psiw, alphaw,
                 dpt1w, dpt1b, dpt2w, dpt2b,
                 idx_out, pd_out, s_out, a_out):
    xb = x_ref[0]
    sq = (jnp.sum(xb * xb, axis=1, keepdims=True)
          + jnp.sum(xall_ref[0] * xall_ref[0], axis=1)[None, :])
    d = sq - _mmT(xb + xb, xall_ref.at[0])

    fiota = lax.broadcasted_iota(jnp.int32, d.shape, 1).astype(jnp.float32)
    cols = []
    for _ in range(KNN):
        m = jnp.min(d, axis=1, keepdims=True)
        ii = jnp.min(jnp.where(d == m, fiota, jnp.float32(NPTS)), axis=1)
        cols.append(ii)
        d = jnp.where(fiota == ii[:, None], jnp.float32(jnp.inf), d)
    idx_out[...] = jnp.stack(cols, axis=1).astype(jnp.int32)

    f = _mmT(inf_ref[0], fc1w) + fc1b[...]
    dr = jnp.maximum(_mmT(f, dpt1w) + dpt1b[...], 0.0)
    dr = _mmT(dr, dpt2w) + dpt2b[...]
    pd_out[...] = _mmT(f, phiw) + dr
    s_out[...] = _pack_bf16(_mmT(f, psiw))
    a_out[...] = _pack_bf16(_mmT(f, alphaw))


def _sc_gather_body(idx_hbm, s_hbm, a_hbm, x_hbm,
                    sg_out, ag_out, xg_out,
                    idx_v, sbuf0, abuf0, xbuf0, sbuf1, abuf1, xbuf1,
                    sem_s0, sem_a0, sem_x0, sem_s1, sem_a1, sem_x1):
    wid = lax.axis_index("s") * SC_CORES + lax.axis_index("c")
    base = wid * PAIRS_PER_WORKER
    pltpu.sync_copy(idx_hbm.at[pl.ds(base, PAIRS_PER_WORKER)], idx_v)

    bufs = ((sbuf0, abuf0, xbuf0), (sbuf1, abuf1, xbuf1))
    sems = ((sem_s0, sem_a0, sem_x0), (sem_s1, sem_a1, sem_x1))

    def issue(c):
        sb, ab, xb = bufs[c % 2]
        ss, sa, sx = sems[c % 2]
        isl = idx_v.at[pl.ds(c * GATHER_CHUNK, GATHER_CHUNK)]
        return (pltpu.async_copy(s_hbm.at[isl], sb, ss),
                pltpu.async_copy(a_hbm.at[isl], ab, sa),
                pltpu.async_copy(x_hbm.at[isl], xb, sx))

    cps = issue(0)
    for c in range(N_CHUNKS):
        nxt = issue(c + 1) if c + 1 < N_CHUNKS else None
        sb, ab, xb = bufs[c % 2]
        off = base + c * GATHER_CHUNK
        cps[0].wait()
        pltpu.sync_copy(sb, sg_out.at[pl.ds(off, GATHER_CHUNK)])
        cps[1].wait()
        pltpu.sync_copy(ab, ag_out.at[pl.ds(off, GATHER_CHUNK)])
        cps[2].wait()
        pltpu.sync_copy(xb, xg_out.at[pl.ds(off, GATHER_CHUNK)])
        cps = nxt


def _stage3_body(xn_ref, inf_ref, pd_ref, sg_ref, ag_ref, xg_ref,
                 del1p, del1b, del2w, del2b,
                 gam1wlo, gam1whi, gam1b, gam2w, gam2b,
                 fc2wlo, fc2whi, fc2b, out_ref):
    nb = ROWS_C
    xn = xn_ref[0]
    xg = xg_ref[...]
    xdiff = (xn[None, :, :] - xg).reshape(KNN * nb, XWIDE)
    h = _mmT(xdiff, del1p) + del1b[...]
    u = jnp.maximum(h, 0.0)
    delta = _mmTb(u, del2w) + del2b[...]
    delta3 = delta.reshape(KNN, nb, DIM)

    pd = pd_ref[...]
    slo, shi = _unpack_bf16(sg_ref[...])
    t_lo = (pd[None, :, :HDIM] - slo + delta3[:, :, :HDIM]).reshape(KNN * nb, HDIM)
    t_hi = (pd[None, :, HDIM:] - shi + delta3[:, :, HDIM:]).reshape(KNN * nb, HDIM)
    g1 = jnp.maximum(_mmTb(t_lo, gam1wlo) + _mmTb(t_hi, gam1whi) + gam1b[...], 0.0)
    gamma = _mmTb(g1, gam2w) + gam2b[...]

    s = gamma.reshape(KNN, nb, DIM)
    smax = jnp.max(s, axis=0, keepdims=True)
    e = jnp.exp(s - smax)
    rinv = 1.0 / jnp.sum(e, axis=0)

    alo, ahi = _unpack_bf16(ag_ref[...])
    y_lo = jnp.sum(e[:, :, :HDIM] * (alo + delta3[:, :, :HDIM]), axis=0) * rinv[:, :HDIM]
    y_hi = jnp.sum(e[:, :, HDIM:] * (ahi + delta3[:, :, HDIM:]), axis=0) * rinv[:, HDIM:]
    out_ref[...] = (_mmT(y_lo, fc2wlo) + _mmT(y_hi, fc2whi)
                    + fc2b[...] + inf_ref[0])


def _full(shape):
    nd = len(shape)
    return pl.BlockSpec(shape, lambda i: (0,) * nd)


def kernel(x, in_f, fc1_w, fc1_b, fc2_w, fc2_b, phi_w, psi_w, alpha_w,
           dpt1_w, dpt1_b, dpt2_w, dpt2_b, gam1_w, gam1_b, gam2_w, gam2_b,
           del1_w, del1_b, del2_w, del2_b):
    f32 = jnp.float32
    xpad = jnp.pad(x, ((0, 0), (0, 0), (0, XPAD - 3)))
    xw = jnp.pad(x, ((0, 0), (0, 0), (0, XWIDE - 3)))
    del1p = jnp.pad(del1_w, ((0, 0), (0, XWIDE - 3)))

    row = lambda v: v.reshape(1, -1)

    def stage1(b):
        return pl.pallas_call(
            _stage1_body,
            grid=(NPTS // ROWS_A,),
            in_specs=[
                pl.BlockSpec((1, ROWS_A, XPAD), lambda i: (b, i, 0)),
                pl.BlockSpec((1, NPTS, XPAD), lambda i: (b, 0, 0)),
                pl.BlockSpec((1, ROWS_A, PDIM), lambda i: (b, i, 0)),
                _full((DIM, PDIM)), _full((1, DIM)),
                _full((DIM, DIM)), _full((DIM, DIM)), _full((DIM, DIM)),
                _full((DIM, DIM)), _full((1, DIM)),
                _full((DIM, DIM)), _full((1, DIM)),
            ],
            out_specs=[
                pl.BlockSpec((ROWS_A, KNN), lambda i: (i, 0)),
                pl.BlockSpec((ROWS_A, DIM), lambda i: (i, 0)),
                pl.BlockSpec((ROWS_A, HDIM), lambda i: (i, 0)),
                pl.BlockSpec((ROWS_A, HDIM), lambda i: (i, 0)),
            ],
            out_shape=[
                jax.ShapeDtypeStruct((NPTS, KNN), jnp.int32),
                jax.ShapeDtypeStruct((NPTS, DIM), f32),
                jax.ShapeDtypeStruct((NPTS, HDIM), jnp.int32),
                jax.ShapeDtypeStruct((NPTS, HDIM), jnp.int32),
            ],
        )(xpad, xpad, in_f,
          fc1_w, row(fc1_b), phi_w, psi_w, alpha_w,
          dpt1_w, row(dpt1_b), dpt2_w, row(dpt2_b))

    sc_gather = functools.partial(
        pl.kernel,
        out_type=[
            jax.ShapeDtypeStruct((PAIRS, HDIM), jnp.int32),
            jax.ShapeDtypeStruct((PAIRS, HDIM), jnp.int32),
            jax.ShapeDtypeStruct((PAIRS, XWIDE), f32),
        ],
        mesh=plsc.VectorSubcoreMesh(core_axis_name="c", subcore_axis_name="s"),
        scratch_types=[
            pltpu.VMEM((PAIRS_PER_WORKER,), jnp.int32),
            pltpu.VMEM((GATHER_CHUNK, HDIM), jnp.int32),
            pltpu.VMEM((GATHER_CHUNK, HDIM), jnp.int32),
            pltpu.VMEM((GATHER_CHUNK, XWIDE), f32),
            pltpu.VMEM((GATHER_CHUNK, HDIM), jnp.int32),
            pltpu.VMEM((GATHER_CHUNK, HDIM), jnp.int32),
            pltpu.VMEM((GATHER_CHUNK, XWIDE), f32),
            pltpu.SemaphoreType.DMA,
            pltpu.SemaphoreType.DMA,
            pltpu.SemaphoreType.DMA,
            pltpu.SemaphoreType.DMA,
            pltpu.SemaphoreType.DMA,
            pltpu.SemaphoreType.DMA,
        ],
    )(_sc_gather_body)

    def stage3(b, h, pdt, sg, ag, xg):
        hb = h * (HALF // ROWS_C)
        return pl.pallas_call(
            _stage3_body,
            grid=(HALF // ROWS_C,),
            in_specs=[
                pl.BlockSpec((1, ROWS_C, XWIDE), lambda i: (b, hb + i, 0)),
                pl.BlockSpec((1, ROWS_C, PDIM), lambda i: (b, hb + i, 0)),
                pl.BlockSpec((ROWS_C, DIM), lambda i: (hb + i, 0)),
                pl.BlockSpec((KNN, ROWS_C, HDIM), lambda i: (0, i, 0)),
                pl.BlockSpec((KNN, ROWS_C, HDIM), lambda i: (0, i, 0)),
                pl.BlockSpec((KNN, ROWS_C, XWIDE), lambda i: (0, i, 0)),
                _full((DIM, XWIDE)), _full((1, DIM)),
                _full((DIM, DIM)), _full((1, DIM)),
                _full((DIM, HDIM)), _full((DIM, HDIM)), _full((1, DIM)),
                _full((DIM, DIM)), _full((1, DIM)),
                _full((PDIM, HDIM)), _full((PDIM, HDIM)), _full((1, PDIM)),
            ],
            out_specs=pl.BlockSpec((ROWS_C, PDIM), lambda i: (i, 0)),
            out_shape=jax.ShapeDtypeStruct((HALF, PDIM), f32),
        )(xw, in_f, pdt,
          sg.reshape(KNN, HALF, HDIM),
          ag.reshape(KNN, HALF, HDIM),
          xg.reshape(KNN, HALF, XWIDE),
          del1p, row(del1_b), del2_w.astype(jnp.bfloat16), row(del2_b),
          gam1_w[:, :HDIM].astype(jnp.bfloat16),
          gam1_w[:, HDIM:].astype(jnp.bfloat16), row(gam1_b),
          (gam2_w * (1.0 / (KNN ** 0.5))).astype(jnp.bfloat16),
          row(gam2_b * (1.0 / (KNN ** 0.5))),
          fc2_w[:, :HDIM], fc2_w[:, HDIM:], row(fc2_b))

    gathered = {}
    tabs = [None, None]
    for b in range(BATCH):
        idx, pdt, st, at = stage1(b)
        tabs[b] = pdt
        for h in range(2):
            idx_h = lax.slice_in_dim(idx, h * HALF, (h + 1) * HALF)
            idx_km = jnp.transpose(idx_h, (1, 0)).reshape(PAIRS)
            gathered[(b, h)] = sc_gather(idx_km, st, at, xw[b])
    outs = []
    for b in range(BATCH):
        halves = []
        for h in range(2):
            sg, ag, xg = gathered[(b, h)]
            halves.append(stage3(b, h, tabs[b], sg, ag, xg))
        outs.append(jnp.concatenate(halves, axis=0))
    return jnp.stack(outs)


# speedup vs baseline: 1.0065x; 1.0065x over previous
import functools

import jax
import jax.numpy as jnp
from jax import lax
from jax.experimental import pallas as pl
from jax.experimental.pallas import tpu as pltpu
from jax.experimental.pallas import tpu_sc as plsc

BATCH = 2
NPTS = 2048
KNN = 16
DIM = 256
HDIM = DIM // 2
PDIM = 64
XPAD = 16
XWIDE = 128

ROWS_A = 512
ROWS_C = 256

SC_CORES = 2
SC_SUBCORES = 16
SC_WORKERS = SC_CORES * SC_SUBCORES
HALF = NPTS // 2
PAIRS = HALF * KNN
PAIRS_PER_WORKER = PAIRS // SC_WORKERS
GATHER_CHUNK = 64
N_CHUNKS = PAIRS_PER_WORKER // GATHER_CHUNK

_NT = (((1,), (1,)), ((), ()))


def _mmT(a, b_ref):
    return lax.dot_general(a, b_ref[...], dimension_numbers=_NT,
                           preferred_element_type=jnp.float32)


def _mmTb(a, b_ref):
    return lax.dot_general(a.astype(jnp.bfloat16), b_ref[...],
                           dimension_numbers=_NT,
                           preferred_element_type=jnp.float32)


def _pack_bf16(v):
    lo = lax.bitcast_convert_type(v[:, :HDIM].astype(jnp.bfloat16), jnp.uint16)
    hi = lax.bitcast_convert_type(v[:, HDIM:].astype(jnp.bfloat16), jnp.uint16)
    word = lo.astype(jnp.uint32) | (hi.astype(jnp.uint32) << 16)
    return lax.bitcast_convert_type(word, jnp.int32)


def _unpack_bf16(w):
    lo = lax.bitcast_convert_type(w << 16, jnp.float32)
    hi = lax.bitcast_convert_type(w & jnp.int32(-65536), jnp.float32)
    return lo, hi


def _stage1_body(x_ref, xall_ref, inf_ref,
                 fc1w, fc1b, phiw, psiw, alphaw,
                 dpt1w, dpt1b, dpt2w, dpt2b,
                 idx_out, pd_out, s_out, a_out):
    xb = x_ref[0]
    d = -2.0 * _mmT(xb, xall_ref.at[0])
    d = d + jnp.sum(xb * xb, axis=1, keepdims=True)
    d = d + jnp.sum(xall_ref[0] * xall_ref[0], axis=1)[None, :]

    fiota = lax.broadcasted_iota(jnp.int32, d.shape, 1).astype(jnp.float32)
    cols = []
    for _ in range(KNN):
        m = jnp.min(d, axis=1, keepdims=True)
        ii = jnp.min(jnp.where(d == m, fiota, jnp.float32(NPTS)), axis=1)
        cols.append(ii)
        d = jnp.where(fiota == ii[:, None], jnp.float32(jnp.inf), d)
    idx_out[...] = jnp.stack(cols, axis=1).astype(jnp.int32)

    f = _mmT(inf_ref[0], fc1w) + fc1b[...]
    dr = jnp.maximum(_mmT(f, dpt1w) + dpt1b[...], 0.0)
    dr = _mmT(dr, dpt2w) + dpt2b[...]
    pd_out[...] = _mmT(f, phiw) + dr
    s_out[...] = _pack_bf16(_mmT(f, psiw))
    a_out[...] = _pack_bf16(_mmT(f, alphaw))


def _sc_gather_body(idx_hbm, s_hbm, a_hbm, x_hbm,
                    sg_out, ag_out, xg_out,
                    idx_v, sbuf0, abuf0, xbuf0, sbuf1, abuf1, xbuf1,
                    sem_s0, sem_a0, sem_x0, sem_s1, sem_a1, sem_x1):
    wid = lax.axis_index("s") * SC_CORES + lax.axis_index("c")
    base = wid * PAIRS_PER_WORKER
    pltpu.sync_copy(idx_hbm.at[pl.ds(base, PAIRS_PER_WORKER)], idx_v)

    bufs = ((sbuf0, abuf0, xbuf0), (sbuf1, abuf1, xbuf1))
    sems = ((sem_s0, sem_a0, sem_x0), (sem_s1, sem_a1, sem_x1))

    def issue(c):
        sb, ab, xb = bufs[c % 2]
        ss, sa, sx = sems[c % 2]
        isl = idx_v.at[pl.ds(c * GATHER_CHUNK, GATHER_CHUNK)]
        return (pltpu.async_copy(s_hbm.at[isl], sb, ss),
                pltpu.async_copy(a_hbm.at[isl], ab, sa),
                pltpu.async_copy(x_hbm.at[isl], xb, sx))

    cps = issue(0)
    for c in range(N_CHUNKS):
        nxt = issue(c + 1) if c + 1 < N_CHUNKS else None
        sb, ab, xb = bufs[c % 2]
        off = base + c * GATHER_CHUNK
        cps[0].wait()
        pltpu.sync_copy(sb, sg_out.at[pl.ds(off, GATHER_CHUNK)])
        cps[1].wait()
        pltpu.sync_copy(ab, ag_out.at[pl.ds(off, GATHER_CHUNK)])
        cps[2].wait()
        pltpu.sync_copy(xb, xg_out.at[pl.ds(off, GATHER_CHUNK)])
        cps = nxt


def _stage3_body(xn_ref, inf_ref, pd_ref, sg_ref, ag_ref, xg_ref,
                 del1p, del1b, del2w, del2b,
                 gam1wlo, gam1whi, gam1b, gam2w, gam2b,
                 fc2wlo, fc2whi, fc2b, out_ref):
    nb = ROWS_C
    xn = xn_ref[0]
    xg = xg_ref[...]
    xdiff = (xn[None, :, :] - xg).reshape(KNN * nb, XWIDE)
    h = _mmT(xdiff, del1p) + del1b[...]
    u = jnp.maximum(h, 0.0)
    delta = _mmTb(u, del2w) + del2b[...]
    delta3 = delta.reshape(KNN, nb, DIM)

    pd = pd_ref[...]
    slo, shi = _unpack_bf16(sg_ref[...])
    t_lo = (pd[None, :, :HDIM] - slo + delta3[:, :, :HDIM]).reshape(KNN * nb, HDIM)
    t_hi = (pd[None, :, HDIM:] - shi + delta3[:, :, HDIM:]).reshape(KNN * nb, HDIM)
    g1 = jnp.maximum(_mmTb(t_lo, gam1wlo) + _mmTb(t_hi, gam1whi) + gam1b[...], 0.0)
    gamma = _mmTb(g1, gam2w) + gam2b[...]

    s = gamma.reshape(KNN, nb, DIM) * (1.0 / (KNN ** 0.5))
    smax = jnp.max(s, axis=0, keepdims=True)
    e = jnp.exp(s - smax)
    rinv = 1.0 / jnp.sum(e, axis=0)

    alo, ahi = _unpack_bf16(ag_ref[...])
    y_lo = jnp.sum(e[:, :, :HDIM] * (alo + delta3[:, :, :HDIM]), axis=0) * rinv[:, :HDIM]
    y_hi = jnp.sum(e[:, :, HDIM:] * (ahi + delta3[:, :, HDIM:]), axis=0) * rinv[:, HDIM:]
    out_ref[...] = (_mmT(y_lo, fc2wlo) + _mmT(y_hi, fc2whi)
                    + fc2b[...] + inf_ref[0])


def _full(shape):
    nd = len(shape)
    return pl.BlockSpec(shape, lambda i: (0,) * nd)


def kernel(x, in_f, fc1_w, fc1_b, fc2_w, fc2_b, phi_w, psi_w, alpha_w,
           dpt1_w, dpt1_b, dpt2_w, dpt2_b, gam1_w, gam1_b, gam2_w, gam2_b,
           del1_w, del1_b, del2_w, del2_b):
    f32 = jnp.float32
    xpad = jnp.pad(x, ((0, 0), (0, 0), (0, XPAD - 3)))
    xw = jnp.pad(x, ((0, 0), (0, 0), (0, XWIDE - 3)))
    del1p = jnp.pad(del1_w, ((0, 0), (0, XWIDE - 3)))

    row = lambda v: v.reshape(1, -1)

    def stage1(b):
        return pl.pallas_call(
            _stage1_body,
            grid=(NPTS // ROWS_A,),
            in_specs=[
                pl.BlockSpec((1, ROWS_A, XPAD), lambda i: (b, i, 0)),
                pl.BlockSpec((1, NPTS, XPAD), lambda i: (b, 0, 0)),
                pl.BlockSpec((1, ROWS_A, PDIM), lambda i: (b, i, 0)),
                _full((DIM, PDIM)), _full((1, DIM)),
                _full((DIM, DIM)), _full((DIM, DIM)), _full((DIM, DIM)),
                _full((DIM, DIM)), _full((1, DIM)),
                _full((DIM, DIM)), _full((1, DIM)),
            ],
            out_specs=[
                pl.BlockSpec((ROWS_A, KNN), lambda i: (i, 0)),
                pl.BlockSpec((ROWS_A, DIM), lambda i: (i, 0)),
                pl.BlockSpec((ROWS_A, HDIM), lambda i: (i, 0)),
                pl.BlockSpec((ROWS_A, HDIM), lambda i: (i, 0)),
            ],
            out_shape=[
                jax.ShapeDtypeStruct((NPTS, KNN), jnp.int32),
                jax.ShapeDtypeStruct((NPTS, DIM), f32),
                jax.ShapeDtypeStruct((NPTS, HDIM), jnp.int32),
                jax.ShapeDtypeStruct((NPTS, HDIM), jnp.int32),
            ],
        )(xpad, xpad, in_f,
          fc1_w, row(fc1_b), phi_w, psi_w, alpha_w,
          dpt1_w, row(dpt1_b), dpt2_w, row(dpt2_b))

    sc_gather = functools.partial(
        pl.kernel,
        out_type=[
            jax.ShapeDtypeStruct((PAIRS, HDIM), jnp.int32),
            jax.ShapeDtypeStruct((PAIRS, HDIM), jnp.int32),
            jax.ShapeDtypeStruct((PAIRS, XWIDE), f32),
        ],
        mesh=plsc.VectorSubcoreMesh(core_axis_name="c", subcore_axis_name="s"),
        scratch_types=[
            pltpu.VMEM((PAIRS_PER_WORKER,), jnp.int32),
            pltpu.VMEM((GATHER_CHUNK, HDIM), jnp.int32),
            pltpu.VMEM((GATHER_CHUNK, HDIM), jnp.int32),
            pltpu.VMEM((GATHER_CHUNK, XWIDE), f32),
            pltpu.VMEM((GATHER_CHUNK, HDIM), jnp.int32),
            pltpu.VMEM((GATHER_CHUNK, HDIM), jnp.int32),
            pltpu.VMEM((GATHER_CHUNK, XWIDE), f32),
            pltpu.SemaphoreType.DMA,
            pltpu.SemaphoreType.DMA,
            pltpu.SemaphoreType.DMA,
            pltpu.SemaphoreType.DMA,
            pltpu.SemaphoreType.DMA,
            pltpu.SemaphoreType.DMA,
        ],
    )(_sc_gather_body)

    def stage3(b, h, pdt, sg, ag, xg):
        hb = h * (HALF // ROWS_C)
        return pl.pallas_call(
            _stage3_body,
            grid=(HALF // ROWS_C,),
            in_specs=[
                pl.BlockSpec((1, ROWS_C, XWIDE), lambda i: (b, hb + i, 0)),
                pl.BlockSpec((1, ROWS_C, PDIM), lambda i: (b, hb + i, 0)),
                pl.BlockSpec((ROWS_C, DIM), lambda i: (hb + i, 0)),
                pl.BlockSpec((KNN, ROWS_C, HDIM), lambda i: (0, i, 0)),
                pl.BlockSpec((KNN, ROWS_C, HDIM), lambda i: (0, i, 0)),
                pl.BlockSpec((KNN, ROWS_C, XWIDE), lambda i: (0, i, 0)),
                _full((DIM, XWIDE)), _full((1, DIM)),
                _full((DIM, DIM)), _full((1, DIM)),
                _full((DIM, HDIM)), _full((DIM, HDIM)), _full((1, DIM)),
                _full((DIM, DIM)), _full((1, DIM)),
                _full((PDIM, HDIM)), _full((PDIM, HDIM)), _full((1, PDIM)),
            ],
            out_specs=pl.BlockSpec((ROWS_C, PDIM), lambda i: (i, 0)),
            out_shape=jax.ShapeDtypeStruct((HALF, PDIM), f32),
        )(xw, in_f, pdt,
          sg.reshape(KNN, HALF, HDIM),
          ag.reshape(KNN, HALF, HDIM),
          xg.reshape(KNN, HALF, XWIDE),
          del1p, row(del1_b), del2_w.astype(jnp.bfloat16), row(del2_b),
          gam1_w[:, :HDIM].astype(jnp.bfloat16),
          gam1_w[:, HDIM:].astype(jnp.bfloat16), row(gam1_b),
          gam2_w.astype(jnp.bfloat16), row(gam2_b),
          fc2_w[:, :HDIM], fc2_w[:, HDIM:], row(fc2_b))

    gathered = {}
    tabs = [None, None]
    for b in range(BATCH):
        idx, pdt, st, at = stage1(b)
        tabs[b] = pdt
        for h in range(2):
            idx_h = lax.slice_in_dim(idx, h * HALF, (h + 1) * HALF)
            idx_km = jnp.transpose(idx_h, (1, 0)).reshape(PAIRS)
            gathered[(b, h)] = sc_gather(idx_km, st, at, xw[b])
    outs = []
    for b in range(BATCH):
        halves = []
        for h in range(2):
            sg, ag, xg = gathered[(b, h)]
            halves.append(stage3(b, h, tabs[b], sg, ag, xg))
        outs.append(jnp.concatenate(halves, axis=0))
    return jnp.stack(outs)
